# Initial kernel scaffold; baseline (speedup 1.0000x reference)
import jax
import jax.numpy as jnp
from jax.experimental import pallas as pl


def kernel(x, reference_points):
    raise NotImplementedError("write your pallas kernel here")


# speedup vs baseline: 7.7683x; 7.7683x over previous
import functools

import jax
import jax.numpy as jnp
from jax.experimental import pallas as pl
from jax.experimental.pallas import tpu as pltpu

K = 5
LANES = 128
NCAND = K * LANES


def _body(x_ref, refT_ref, out_ref, xn2_ref, m_ref, *, bq, br, nr):
    r = pl.program_id(1)

    @pl.when(r == 0)
    def _init():
        xv = x_ref[...]
        nrm = jnp.sqrt(jnp.sum(xv * xv, axis=1, keepdims=True))
        xn = xv / jnp.maximum(nrm, 1e-12)
        xn2_ref[...] = -2.0 * xn
        m_ref[...] = jnp.full((bq, NCAND), jnp.inf, jnp.float32)

    rT = refT_ref[...]
    rsq = jnp.sum(rT * rT, axis=0, keepdims=True)
    xn2 = xn2_ref[...]
    dot = jax.lax.dot_general(
        xn2, rT, (((1,), (0,)), ((), ())),
        preferred_element_type=jnp.float32)
    s = rsq + dot

    m = [m_ref[:, k * LANES:(k + 1) * LANES] for k in range(K)]
    for c in range(br // LANES):
        v = s[:, c * LANES:(c + 1) * LANES]
        for k in range(K):
            lo = jnp.minimum(m[k], v)
            v = jnp.maximum(m[k], v)
            m[k] = lo
    for k in range(K):
        m_ref[:, k * LANES:(k + 1) * LANES] = m[k]

    @pl.when(r == nr - 1)
    def _final():
        xn2v = xn2_ref[...]
        xsq = 0.25 * jnp.sum(xn2v * xn2v, axis=1, keepdims=True)
        cand = m_ref[...]
        lane = jax.lax.broadcasted_iota(jnp.int32, (bq, NCAND), 1)
        ws = []
        for _ in range(K):
            mn = jnp.min(cand, axis=1, keepdims=True)
            fi = jnp.min(jnp.where(cand == mn, lane, NCAND), axis=1,
                         keepdims=True)
            cand = jnp.where(lane == fi, jnp.inf, cand)
            d = jnp.sqrt(jnp.maximum(xsq + mn, 1e-12))
            ws.append(jnp.exp(-d))
        wsum = jnp.maximum(sum(ws), 1e-12)
        cols = jax.lax.broadcasted_iota(jnp.int32, (bq, LANES), 1)
        acc = jnp.zeros((bq, LANES), jnp.float32)
        for k in range(K):
            acc = jnp.where(cols == k, ws[k] / wsum, acc)
        out_ref[...] = acc


def kernel(x, reference_points):
    q, d = x.shape
    r_tot = reference_points.shape[0]
    assert d == LANES
    bq = min(1024, q)
    nq = q // bq
    br = 1024
    nr = (r_tot + br - 1) // br
    rpad = nr * br

    refT = reference_points.T
    if rpad > r_tot:
        pad = jnp.zeros((d, rpad - r_tot), jnp.float32).at[0, :].set(1000.0)
        refT = jnp.concatenate([refT, pad], axis=1)

    out = pl.pallas_call(
        functools.partial(_body, bq=bq, br=br, nr=nr),
        grid=(nq, nr),
        in_specs=[
            pl.BlockSpec((bq, d), lambda qi, ri: (qi, 0)),
            pl.BlockSpec((d, br), lambda qi, ri: (0, ri)),
        ],
        out_specs=pl.BlockSpec((bq, LANES), lambda qi, ri: (qi, 0)),
        out_shape=jax.ShapeDtypeStruct((q, LANES), jnp.float32),
        scratch_shapes=[
            pltpu.VMEM((bq, d), jnp.float32),
            pltpu.VMEM((bq, NCAND), jnp.float32),
        ],
        compiler_params=pltpu.CompilerParams(
            dimension_semantics=("parallel", "arbitrary")),
    )(x, refT)
    return out[:, :K]


# speedup vs baseline: 10.0907x; 1.2990x over previous
import functools

import jax
import jax.numpy as jnp
from jax.experimental import pallas as pl
from jax.experimental.pallas import tpu as pltpu

K = 5
LANES = 128
NREG = 9
NCAND = NREG * LANES


def _ce(a, b):
    return jnp.minimum(a, b), jnp.maximum(a, b)


def _body(x_ref, refT_ref, out_ref, xn2_ref, m_ref, *, bq, br, nr):
    r = pl.program_id(1)

    @pl.when(r == 0)
    def _init():
        xv = x_ref[...]
        nrm = jnp.sqrt(jnp.sum(xv * xv, axis=1, keepdims=True))
        xn = xv / jnp.maximum(nrm, 1e-12)
        xn2_ref[...] = -2.0 * xn
        m_ref[...] = jnp.full((bq, NCAND), jnp.inf, jnp.float32)

    rT = refT_ref[...]
    ones = jnp.ones((1, LANES), jnp.float32)
    rsq = jax.lax.dot_general(
        ones, rT * rT, (((1,), (0,)), ((), ())),
        preferred_element_type=jnp.float32)
    xn2 = xn2_ref[...]
    dot = jax.lax.dot_general(
        xn2, rT, (((1,), (0,)), ((), ())),
        preferred_element_type=jnp.float32)
    s = rsq + dot

    m = [m_ref[:, k * LANES:(k + 1) * LANES] for k in range(NREG)]
    for g in range(br // (4 * LANES)):
        c0 = s[:, (4 * g + 0) * LANES:(4 * g + 1) * LANES]
        c1 = s[:, (4 * g + 1) * LANES:(4 * g + 2) * LANES]
        c2 = s[:, (4 * g + 2) * LANES:(4 * g + 3) * LANES]
        c3 = s[:, (4 * g + 3) * LANES:(4 * g + 4) * LANES]
        c0, c1 = _ce(c0, c1)
        c2, c3 = _ce(c2, c3)
        c0, c2 = _ce(c0, c2)
        c1, c3 = _ce(c1, c3)
        c1, c2 = _ce(c1, c2)
        v = c0
        for k in range(4):
            m[k], v = _ce(m[k], v)
        m[4] = jnp.minimum(m[4], v)
        m[5], v = _ce(m[5], c1)
        m[6] = jnp.minimum(m[6], v)
        m[7] = jnp.minimum(m[7], c2)
        m[8] = jnp.minimum(m[8], c3)
    for k in range(NREG):
        m_ref[:, k * LANES:(k + 1) * LANES] = m[k]

    @pl.when(r == nr - 1)
    def _final():
        xn2v = xn2_ref[...]
        xsq = 0.25 * jnp.sum(xn2v * xn2v, axis=1, keepdims=True)
        cand = m_ref[...]
        lane = jax.lax.broadcasted_iota(jnp.int32, (bq, NCAND), 1)
        ws = []
        for _ in range(K):
            mn = jnp.min(cand, axis=1, keepdims=True)
            fi = jnp.min(jnp.where(cand == mn, lane, NCAND), axis=1,
                         keepdims=True)
            cand = jnp.where(lane == fi, jnp.inf, cand)
            d = jnp.sqrt(jnp.maximum(xsq + mn, 1e-12))
            ws.append(jnp.exp(-d))
        wsum = jnp.maximum(sum(ws), 1e-12)
        cols = jax.lax.broadcasted_iota(jnp.int32, (bq, LANES), 1)
        acc = jnp.zeros((bq, LANES), jnp.float32)
        for k in range(K):
            acc = jnp.where(cols == k, ws[k] / wsum, acc)
        out_ref[...] = acc


def kernel(x, reference_points):
    q, d = x.shape
    r_tot = reference_points.shape[0]
    assert d == LANES
    bq = min(2048, q)
    nq = q // bq
    br = 2048
    nr = (r_tot + br - 1) // br
    rpad = nr * br

    refT = reference_points.T
    if rpad > r_tot:
        pad = jnp.zeros((d, rpad - r_tot), jnp.float32).at[0, :].set(1000.0)
        refT = jnp.concatenate([refT, pad], axis=1)

    out = pl.pallas_call(
        functools.partial(_body, bq=bq, br=br, nr=nr),
        grid=(nq, nr),
        in_specs=[
            pl.BlockSpec((bq, d), lambda qi, ri: (qi, 0)),
            pl.BlockSpec((d, br), lambda qi, ri: (0, ri)),
        ],
        out_specs=pl.BlockSpec((bq, LANES), lambda qi, ri: (qi, 0)),
        out_shape=jax.ShapeDtypeStruct((q, LANES), jnp.float32),
        scratch_shapes=[
            pltpu.VMEM((bq, d), jnp.float32),
            pltpu.VMEM((bq, NCAND), jnp.float32),
        ],
        compiler_params=pltpu.CompilerParams(
            dimension_semantics=("parallel", "arbitrary")),
    )(x, refT)
    return out[:, :K]
